# baseline (speedup 1.0000x reference)
import functools

import jax
import jax.numpy as jnp
from jax.experimental import pallas as pl
from jax.experimental.pallas import tpu as pltpu

_NSPLIT = 4


def _video_copy(x_hbm, scr, sem, video, slot, s, nsplit):
    sk = s // nsplit
    return [
        pltpu.make_async_copy(
            x_hbm.at[video, pl.ds(j * sk, sk)],
            scr.at[slot, pl.ds(j * sk, sk)],
            sem.at[slot, j],
        )
        for j in range(nsplit)
    ]


def _fused_body(seg_ref, x_hbm, c1_ref, c2_ref, Wenc_ref, benc_ref,
                Wt1_ref, bt1_ref, Wt2_ref, bt2_ref, temb_ref, Wc_ref, bc_ref,
                out_ref, scr, sem, *, s, f, tn, nb, k):
    b = pl.program_id(0)
    slot = jax.lax.rem(b, 2)
    sk = s // k
    d = x_hbm.shape[-1]
    TE = temb_ref.shape[1]

    @pl.when(b == 0)
    def _first_fetch():
        for c in _video_copy(x_hbm, scr, sem, 0, 0, s, _NSPLIT):
            c.start()

    @pl.when(b + 1 < nb)
    def _prefetch_next():
        for c in _video_copy(x_hbm, scr, sem, b + 1,
                             jax.lax.rem(b + 1, 2), s, _NSPLIT):
            c.start()

    for c in _video_copy(x_hbm, scr, sem, b, slot, s, _NSPLIT):
        c.wait()

    f_lane = jax.lax.rem(jax.lax.broadcasted_iota(jnp.int32, (1, sk * f), 1), f)
    sums = jnp.zeros((1, d), jnp.float32)
    for kk in range(k):
        x = scr[slot, kk * sk:(kk + 1) * sk].reshape(sk * f, d)
        l_lane = jnp.concatenate(
            [jnp.full((1, f), seg_ref[b, kk * sk + j], jnp.int32)
             for j in range(sk)], axis=1)
        mask = (f_lane < l_lane).astype(jnp.float32)
        sums = sums + jnp.dot(mask, x, preferred_element_type=jnp.float32)
    count = jnp.float32(0)
    for j in range(s):
        count = count + seg_ref[b, j].astype(jnp.float32)
    pooled = sums / count
    vf = jax.nn.relu(
        jnp.dot(pooled, Wenc_ref[...], preferred_element_type=jnp.float32)
        + benc_ref[...])
    cat = jnp.concatenate([c1_ref[0], c2_ref[0], vf], axis=1)
    h = jax.nn.relu(
        jnp.dot(cat, Wt1_ref[...], preferred_element_type=jnp.float32)
        + bt1_ref[...])
    logits = (jnp.dot(h, Wt2_ref[...], preferred_element_type=jnp.float32)
              + bt2_ref[...])
    m = jnp.max(logits, axis=1, keepdims=True)
    e = jnp.exp(logits - m)
    tp = e / jnp.sum(e, axis=1, keepdims=True)
    E_T = jax.lax.dot_general(Wc_ref[0:TE, :], temb_ref[...],
                              dimension_numbers=(((0,), (1,)), ((), ())),
                              preferred_element_type=jnp.float32)
    catv = jnp.concatenate([vf, c1_ref[0], c2_ref[0]], axis=1)
    V = jax.lax.dot_general(Wc_ref[TE:, :], catv,
                            dimension_numbers=(((0,), (1,)), ((), ())),
                            preferred_element_type=jnp.float32)
    q = jax.nn.relu(E_T + V + bc_ref[...]).astype(jnp.bfloat16)

    f_sub = jax.lax.rem(jax.lax.broadcasted_iota(jnp.int32, (sk * f, 1), 0), f)
    for kk in range(k):
        x = scr[slot, kk * sk:(kk + 1) * sk].reshape(sk * f, d)
        L = jnp.dot(x.astype(jnp.bfloat16), q,
                    preferred_element_type=jnp.float32)
        sc = jax.nn.sigmoid(L) * tp
        sc = jax.nn.relu(sc - 0.01)
        tot = jnp.sum(sc, axis=1, keepdims=True) * (1.0 / tn)
        l_sub = jnp.concatenate(
            [jnp.full((f, 1), seg_ref[b, kk * sk + j], jnp.int32)
             for j in range(sk)], axis=0)
        tot = jnp.where(f_sub < l_sub, tot, 0.0)
        out_ref[0, kk * sk:(kk + 1) * sk] = tot.reshape(sk, f)


def kernel(batch, seg_len, concept1, concept2, W_enc, b_enc, W_t1, b_t1,
           W_t2, b_t2, topic_emb, W_c, b_c):
    B, S, F, D = batch.shape
    TN, TE = topic_emb.shape
    SH = W_enc.shape[1]
    CD = concept1.shape[1]
    K = 4 if S % 4 == 0 else 1

    seg_len = seg_len.astype(jnp.int32)

    const = lambda *idx: (lambda b, seg: idx)

    overall = pl.pallas_call(
        functools.partial(_fused_body, s=S, f=F, tn=float(TN), nb=B, k=K),
        grid_spec=pltpu.PrefetchScalarGridSpec(
            num_scalar_prefetch=1,
            grid=(B,),
            in_specs=[
                pl.BlockSpec(memory_space=pl.ANY),
                pl.BlockSpec((1, 1, CD), lambda b, seg: (b, 0, 0)),
                pl.BlockSpec((1, 1, CD), lambda b, seg: (b, 0, 0)),
                pl.BlockSpec((D, SH), const(0, 0)),
                pl.BlockSpec((1, SH), const(0, 0)),
                pl.BlockSpec(W_t1.shape, const(0, 0)),
                pl.BlockSpec((1, W_t1.shape[1]), const(0, 0)),
                pl.BlockSpec(W_t2.shape, const(0, 0)),
                pl.BlockSpec((1, TN), const(0, 0)),
                pl.BlockSpec((TN, TE), const(0, 0)),
                pl.BlockSpec(W_c.shape, const(0, 0)),
                pl.BlockSpec((D, 1), const(0, 0)),
            ],
            out_specs=pl.BlockSpec((1, S, F), lambda b, seg: (b, 0, 0)),
            scratch_shapes=[
                pltpu.VMEM((2, S, F, D), jnp.float32),
                pltpu.SemaphoreType.DMA((2, _NSPLIT)),
            ],
        ),
        out_shape=jax.ShapeDtypeStruct((B, S, F), jnp.float32),
        compiler_params=pltpu.CompilerParams(
            dimension_semantics=("arbitrary",)),
    )(seg_len, batch, concept1.reshape(B, 1, CD), concept2.reshape(B, 1, CD),
      W_enc, b_enc.reshape(1, SH), W_t1, b_t1.reshape(1, -1),
      W_t2, b_t2.reshape(1, TN), topic_emb, W_c, b_c.reshape(D, 1))

    return (overall, overall)


# speedup vs baseline: 1.1266x; 1.1266x over previous
import functools

import jax
import jax.numpy as jnp
from jax.experimental import pallas as pl
from jax.experimental.pallas import tpu as pltpu

_K = 4


def _fused_body(seg_ref, *refs, s, f, tn, k):
    x_refs = refs[:k]
    (c1_ref, c2_ref, Wenc_ref, benc_ref, Wt1_ref, bt1_ref, Wt2_ref, bt2_ref,
     temb_ref, Wc_ref, bc_ref) = refs[k:k + 11]
    out_ref = refs[k + 11]
    q_scr, tp_scr = refs[k + 12:]
    b = pl.program_id(0)
    p = pl.program_id(1)
    sk = s // k
    d = x_refs[0].shape[-1]

    @pl.when(p == 0)
    def _pool_and_mlp():
        TE = temb_ref.shape[1] - 1
        f_lane = jax.lax.rem(
            jax.lax.broadcasted_iota(jnp.int32, (1, sk * f), 1), f)
        sums = jnp.zeros((1, d), jnp.float32)
        for kk in range(k):
            x = x_refs[kk][0].reshape(sk * f, d)
            l_lane = jnp.concatenate(
                [jnp.full((1, f), seg_ref[b, kk * sk + j], jnp.int32)
                 for j in range(sk)], axis=1)
            mask = (f_lane < l_lane).astype(jnp.float32)
            sums = sums + jnp.dot(mask, x, preferred_element_type=jnp.float32)
        count = jnp.float32(0)
        for j in range(s):
            count = count + seg_ref[b, j].astype(jnp.float32)
        pooled = sums / count
        row = jax.lax.broadcasted_iota(jnp.int32, c1_ref.shape, 0)
        c1 = jnp.sum(jnp.where(row == b, c1_ref[...], 0.0), axis=0,
                     keepdims=True)
        c2 = jnp.sum(jnp.where(row == b, c2_ref[...], 0.0), axis=0,
                     keepdims=True)
        vf = jax.nn.relu(
            jnp.dot(pooled, Wenc_ref[...], preferred_element_type=jnp.float32)
            + benc_ref[...])
        cat = jnp.concatenate([c1, c2, vf], axis=1)
        h = jax.nn.relu(
            jnp.dot(cat, Wt1_ref[...], preferred_element_type=jnp.float32)
            + bt1_ref[...])
        logits = (jnp.dot(h, Wt2_ref[...], preferred_element_type=jnp.float32)
                  + bt2_ref[...])
        m = jnp.max(logits, axis=1, keepdims=True)
        e = jnp.exp(logits - m)
        tp_scr[...] = e / jnp.sum(e, axis=1, keepdims=True)
        Wcb = jnp.concatenate([Wc_ref[0:TE, :], bc_ref[...]], axis=0)
        E_T = jax.lax.dot_general(Wcb, temb_ref[...],
                                  dimension_numbers=(((0,), (1,)), ((), ())),
                                  preferred_element_type=jnp.float32)
        catv = jnp.concatenate([vf, c1, c2], axis=1)
        V = jax.lax.dot_general(Wc_ref[TE:, :], catv,
                                dimension_numbers=(((0,), (1,)), ((), ())),
                                preferred_element_type=jnp.float32)
        q_scr[...] = jax.nn.relu(E_T + V).astype(jnp.bfloat16)

    @pl.when(p == 1)
    def _score():
        q = q_scr[...]
        tp = tp_scr[...]
        f_sub = jax.lax.rem(
            jax.lax.broadcasted_iota(jnp.int32, (sk * f, 1), 0), f)
        for kk in range(k):
            x = x_refs[kk][0].reshape(sk * f, d).astype(jnp.bfloat16)
            L = jnp.dot(x, q, preferred_element_type=jnp.float32)
            sc = jax.nn.sigmoid(L) * tp
            sc = jax.nn.relu(sc - 0.01)
            tot = jnp.sum(sc, axis=1, keepdims=True) * (1.0 / tn)
            l_sub = jnp.concatenate(
                [jnp.full((f, 1), seg_ref[b, kk * sk + j], jnp.int32)
                 for j in range(sk)], axis=0)
            tot = jnp.where(f_sub < l_sub, tot, 0.0)
            out_ref[0, kk * sk:(kk + 1) * sk] = tot.reshape(sk, f)


def kernel(batch, seg_len, concept1, concept2, W_enc, b_enc, W_t1, b_t1,
           W_t2, b_t2, topic_emb, W_c, b_c):
    B, S, F, D = batch.shape
    TN, TE = topic_emb.shape
    SH = W_enc.shape[1]
    CD = concept1.shape[1]
    K = _K if S % _K == 0 else 1
    SK = S // K

    seg_len = seg_len.astype(jnp.int32)

    const = lambda *idx: (lambda b, p, seg: idx)
    x_specs = [
        pl.BlockSpec((1, SK, F, D),
                     lambda b, p, seg, kk=kk: (b, kk, 0, 0))
        for kk in range(K)
    ]

    overall = pl.pallas_call(
        functools.partial(_fused_body, s=S, f=F, tn=float(TN), k=K),
        grid_spec=pltpu.PrefetchScalarGridSpec(
            num_scalar_prefetch=1,
            grid=(B, 2),
            in_specs=x_specs + [
                pl.BlockSpec((B, CD), const(0, 0)),
                pl.BlockSpec((B, CD), const(0, 0)),
                pl.BlockSpec((D, SH), const(0, 0)),
                pl.BlockSpec((1, SH), const(0, 0)),
                pl.BlockSpec(W_t1.shape, const(0, 0)),
                pl.BlockSpec((1, W_t1.shape[1]), const(0, 0)),
                pl.BlockSpec(W_t2.shape, const(0, 0)),
                pl.BlockSpec((1, TN), const(0, 0)),
                pl.BlockSpec((TN, TE + 1), const(0, 0)),
                pl.BlockSpec(W_c.shape, const(0, 0)),
                pl.BlockSpec((1, D), const(0, 0)),
            ],
            out_specs=pl.BlockSpec((1, S, F), lambda b, p, seg: (b, 0, 0)),
            scratch_shapes=[
                pltpu.VMEM((D, TN), jnp.bfloat16),
                pltpu.VMEM((1, TN), jnp.float32),
            ],
        ),
        out_shape=jax.ShapeDtypeStruct((B, S, F), jnp.float32),
        compiler_params=pltpu.CompilerParams(
            dimension_semantics=("parallel", "arbitrary")),
    )(seg_len, *([batch] * K), concept1, concept2,
      W_enc, b_enc.reshape(1, SH), W_t1,
      b_t1.reshape(1, -1), W_t2, b_t2.reshape(1, TN),
      jnp.concatenate([topic_emb, jnp.ones((TN, 1), jnp.float32)], axis=1),
      W_c, b_c.reshape(1, D))

    return (overall, overall)


# speedup vs baseline: 1.1443x; 1.0156x over previous
import functools

import jax
import jax.numpy as jnp
from jax.experimental import pallas as pl
from jax.experimental.pallas import tpu as pltpu

_K = 10


def _fused_body(seg_ref, *refs, s, f, tn, k):
    x_refs = refs[:k]
    (c1_ref, c2_ref, Wenc_ref, benc_ref, Wt1_ref, bt1_ref, Wt2_ref, bt2_ref,
     temb_ref, Wc_ref, bc_ref) = refs[k:k + 11]
    out_ref = refs[k + 11]
    q_scr, tp_scr = refs[k + 12:]
    b = pl.program_id(0)
    p = pl.program_id(1)
    sk = s // k
    d = x_refs[0].shape[-1]

    @pl.when(p == 0)
    def _pool_and_mlp():
        TE = temb_ref.shape[1] - 1
        f_lane = jax.lax.rem(
            jax.lax.broadcasted_iota(jnp.int32, (1, sk * f), 1), f)
        sums = jnp.zeros((1, d), jnp.float32)
        for kk in range(k):
            x = x_refs[kk][0].reshape(sk * f, d)
            l_lane = jnp.concatenate(
                [jnp.full((1, f), seg_ref[b, kk * sk + j], jnp.int32)
                 for j in range(sk)], axis=1)
            mask = (f_lane < l_lane).astype(jnp.float32)
            sums = sums + jnp.dot(mask, x, preferred_element_type=jnp.float32)
        count = jnp.float32(0)
        for j in range(s):
            count = count + seg_ref[b, j].astype(jnp.float32)
        pooled = sums / count
        row = jax.lax.broadcasted_iota(jnp.int32, c1_ref.shape, 0)
        c1 = jnp.sum(jnp.where(row == b, c1_ref[...], 0.0), axis=0,
                     keepdims=True)
        c2 = jnp.sum(jnp.where(row == b, c2_ref[...], 0.0), axis=0,
                     keepdims=True)
        vf = jax.nn.relu(
            jnp.dot(pooled, Wenc_ref[...], preferred_element_type=jnp.float32)
            + benc_ref[...])
        cat = jnp.concatenate([c1, c2, vf], axis=1)
        h = jax.nn.relu(
            jnp.dot(cat, Wt1_ref[...], preferred_element_type=jnp.float32)
            + bt1_ref[...])
        logits = (jnp.dot(h, Wt2_ref[...], preferred_element_type=jnp.float32)
                  + bt2_ref[...])
        m = jnp.max(logits, axis=1, keepdims=True)
        e = jnp.exp(logits - m)
        tp_scr[...] = e / jnp.sum(e, axis=1, keepdims=True)
        Wcb = jnp.concatenate([Wc_ref[0:TE, :], bc_ref[...]], axis=0)
        E_T = jax.lax.dot_general(Wcb, temb_ref[...],
                                  dimension_numbers=(((0,), (1,)), ((), ())),
                                  preferred_element_type=jnp.float32)
        catv = jnp.concatenate([vf, c1, c2], axis=1)
        V = jax.lax.dot_general(Wc_ref[TE:, :], catv,
                                dimension_numbers=(((0,), (1,)), ((), ())),
                                preferred_element_type=jnp.float32)
        q_scr[...] = jax.nn.relu(E_T + V).astype(jnp.bfloat16)

    @pl.when(p == 1)
    def _score():
        q = q_scr[...]
        tp = tp_scr[...]
        f_sub = jax.lax.rem(
            jax.lax.broadcasted_iota(jnp.int32, (sk * f, 1), 0), f)
        for kk in range(k):
            x = x_refs[kk][0].reshape(sk * f, d).astype(jnp.bfloat16)
            L = jnp.dot(x, q, preferred_element_type=jnp.float32)
            sc = jax.nn.sigmoid(L) * tp
            sc = jax.nn.relu(sc - 0.01)
            tot = jnp.sum(sc, axis=1, keepdims=True) * (1.0 / tn)
            l_sub = jnp.concatenate(
                [jnp.full((f, 1), seg_ref[b, kk * sk + j], jnp.int32)
                 for j in range(sk)], axis=0)
            tot = jnp.where(f_sub < l_sub, tot, 0.0)
            out_ref[0, kk * sk:(kk + 1) * sk] = tot.reshape(sk, f)


def kernel(batch, seg_len, concept1, concept2, W_enc, b_enc, W_t1, b_t1,
           W_t2, b_t2, topic_emb, W_c, b_c):
    B, S, F, D = batch.shape
    TN, TE = topic_emb.shape
    SH = W_enc.shape[1]
    CD = concept1.shape[1]
    K = _K if S % _K == 0 else 1
    SK = S // K

    seg_len = seg_len.astype(jnp.int32)

    const = lambda *idx: (lambda b, p, seg: idx)
    x_specs = [
        pl.BlockSpec((1, SK, F, D),
                     lambda b, p, seg, kk=kk: (b, kk, 0, 0))
        for kk in range(K)
    ]

    overall = pl.pallas_call(
        functools.partial(_fused_body, s=S, f=F, tn=float(TN), k=K),
        grid_spec=pltpu.PrefetchScalarGridSpec(
            num_scalar_prefetch=1,
            grid=(B, 2),
            in_specs=x_specs + [
                pl.BlockSpec((B, CD), const(0, 0)),
                pl.BlockSpec((B, CD), const(0, 0)),
                pl.BlockSpec((D, SH), const(0, 0)),
                pl.BlockSpec((1, SH), const(0, 0)),
                pl.BlockSpec(W_t1.shape, const(0, 0)),
                pl.BlockSpec((1, W_t1.shape[1]), const(0, 0)),
                pl.BlockSpec(W_t2.shape, const(0, 0)),
                pl.BlockSpec((1, TN), const(0, 0)),
                pl.BlockSpec((TN, TE + 1), const(0, 0)),
                pl.BlockSpec(W_c.shape, const(0, 0)),
                pl.BlockSpec((1, D), const(0, 0)),
            ],
            out_specs=pl.BlockSpec((1, S, F), lambda b, p, seg: (b, 0, 0)),
            scratch_shapes=[
                pltpu.VMEM((D, TN), jnp.bfloat16),
                pltpu.VMEM((1, TN), jnp.float32),
            ],
        ),
        out_shape=jax.ShapeDtypeStruct((B, S, F), jnp.float32),
        compiler_params=pltpu.CompilerParams(
            dimension_semantics=("parallel", "arbitrary")),
    )(seg_len, *([batch] * K), concept1, concept2,
      W_enc, b_enc.reshape(1, SH), W_t1,
      b_t1.reshape(1, -1), W_t2, b_t2.reshape(1, TN),
      jnp.concatenate([topic_emb, jnp.ones((TN, 1), jnp.float32)], axis=1),
      W_c, b_c.reshape(1, D))

    return (overall, overall)
